# Initial kernel scaffold; baseline (speedup 1.0000x reference)
import functools

import jax
import jax.numpy as jnp
from jax import lax
from jax.experimental import pallas as pl
from jax.experimental.pallas import tpu as pltpu
from jax.experimental.pallas import tpu_sc as plsc

N = 10000
NP = 10240
E = 320000
NC, NS, LANES = 2, 16, 16
C = 128
CH = 160
EPT = CH * C
EP = NS * EPT
RB = 1280


def _sc_mesh():
    return plsc.VectorSubcoreMesh(core_axis_name="c", subcore_axis_name="s")


@functools.lru_cache(maxsize=None)
def _deg_kernel():
    chc = CH // NC

    @functools.partial(
        pl.kernel,
        out_type=jax.ShapeDtypeStruct((NC, NP), jnp.float32),
        mesh=_sc_mesh(),
        scratch_types=[
            pltpu.VMEM((chc, C), jnp.int32),
            pltpu.VMEM((chc, C), jnp.float32),
            pltpu.VMEM_SHARED((NP,), jnp.float32),
        ],
    )
    def deg(dst_hbm, ew_hbm, zeros_hbm, out_hbm, dst_v, ew_v, acc):
        c = lax.axis_index("c")
        s = lax.axis_index("s")
        rps = NP // NS
        pltpu.sync_copy(zeros_hbm.at[pl.ds(s * rps, rps)],
                        acc.at[pl.ds(s * rps, rps)])
        pltpu.sync_copy(dst_hbm.at[s, pl.ds(c * chc, chc)], dst_v)
        pltpu.sync_copy(ew_hbm.at[s, pl.ds(c * chc, chc)], ew_v)
        plsc.subcore_barrier()

        @pl.loop(0, chc)
        def _(j):
            pltpu.sync_copy(ew_v.at[j], acc.at[dst_v.at[j]], add=True)

        plsc.subcore_barrier()

        @pl.when(s == 0)
        def _():
            pltpu.sync_copy(acc, out_hbm.at[c])

    return deg


@functools.lru_cache(maxsize=None)
def _agg_kernel(dh):
    @functools.partial(
        pl.kernel,
        out_type=jax.ShapeDtypeStruct((NC, NP, dh), jnp.float32),
        mesh=_sc_mesh(),
        scratch_types=[
            pltpu.VMEM((CH, C), jnp.int32),
            pltpu.VMEM((CH, C), jnp.int32),
            pltpu.VMEM((CH, C), jnp.float32),
            pltpu.VMEM((C, dh), jnp.float32),
            pltpu.VMEM_SHARED((NP, dh), jnp.float32),
            pltpu.SemaphoreType.DMA,
        ],
    )
    def agg(hp_hbm, src_hbm, dst_hbm, ew_hbm, zeros_hbm, out_hbm,
            src_v, dst_v, ew_v, rows_v, acc, sem):
        c = lax.axis_index("c")
        s = lax.axis_index("s")
        rps = NP // NS
        pltpu.sync_copy(zeros_hbm.at[pl.ds(s * rps, rps)],
                        acc.at[pl.ds(s * rps, rps)])
        pltpu.sync_copy(src_hbm.at[s], src_v)
        pltpu.sync_copy(dst_hbm.at[s], dst_v)
        pltpu.sync_copy(ew_hbm.at[s], ew_v)
        plsc.subcore_barrier()

        @pl.loop(0, CH)
        def _(j):
            pltpu.async_copy(hp_hbm.at[c].at[src_v.at[j]], rows_v, sem).wait()

            @pl.loop(0, C)
            def _(r):
                coef = plsc.load_gather(
                    ew_v, [jnp.full((LANES,), j, jnp.int32),
                           jnp.full((LANES,), r, jnp.int32)])
                for k in range(dh // LANES):
                    sl = pl.ds(k * LANES, LANES)
                    rows_v[r, sl] = rows_v[r, sl] * coef

            pltpu.sync_copy(rows_v, acc.at[dst_v.at[j]], add=True)

        plsc.subcore_barrier()

        @pl.when(s == 0)
        def _():
            pltpu.sync_copy(acc, out_hbm.at[c])

    return agg


def _tc_first(degT, x, w1):
    d = w1.shape[1]
    dh = d // 2

    def body(deg_ref, x_ref, w_ref, dinv_ref, hp_ref):
        deg = deg_ref[:, 0] + deg_ref[:, 1] + 1.0
        dinv = jnp.where(deg > 0, lax.rsqrt(deg), 0.0)[:, None]
        dinv_ref[...] = dinv
        h = jnp.dot(x_ref[...], w_ref[...],
                    preferred_element_type=jnp.float32) * dinv
        hp_ref[0] = h[:, :dh]
        hp_ref[1] = h[:, dh:]

    return pl.pallas_call(
        body,
        grid=(NP // RB,),
        in_specs=[
            pl.BlockSpec((RB, 2), lambda i: (i, 0)),
            pl.BlockSpec((RB, x.shape[1]), lambda i: (i, 0)),
            pl.BlockSpec(w1.shape, lambda i: (0, 0)),
        ],
        out_specs=[
            pl.BlockSpec((RB, 1), lambda i: (i, 0)),
            pl.BlockSpec((NC, RB, dh), lambda i: (0, i, 0)),
        ],
        out_shape=[
            jax.ShapeDtypeStruct((NP, 1), jnp.float32),
            jax.ShapeDtypeStruct((NC, NP, dh), jnp.float32),
        ],
    )(degT, x, w1)


def _tc_mid(agg, hp, dinv, b2d, wn):
    d = b2d.shape[1]
    dh = d // 2
    dn = wn.shape[1]
    dnh = dn // 2

    def body(a_ref, hp_ref, dinv_ref, b_ref, w_ref, out_ref):
        z = jnp.concatenate([a_ref[0] + hp_ref[0], a_ref[1] + hp_ref[1]],
                            axis=1)
        dinv = dinv_ref[...]
        act = jnp.maximum(z * dinv + b_ref[...], 0.0)
        hn = jnp.dot(act, w_ref[...],
                     preferred_element_type=jnp.float32) * dinv
        out_ref[0] = hn[:, :dnh]
        out_ref[1] = hn[:, dnh:]

    return pl.pallas_call(
        body,
        grid=(NP // RB,),
        in_specs=[
            pl.BlockSpec((NC, RB, dh), lambda i: (0, i, 0)),
            pl.BlockSpec((NC, RB, dh), lambda i: (0, i, 0)),
            pl.BlockSpec((RB, 1), lambda i: (i, 0)),
            pl.BlockSpec((1, d), lambda i: (0, 0)),
            pl.BlockSpec(wn.shape, lambda i: (0, 0)),
        ],
        out_specs=pl.BlockSpec((NC, RB, dnh), lambda i: (0, i, 0)),
        out_shape=jax.ShapeDtypeStruct((NC, NP, dnh), jnp.float32),
    )(agg, hp, dinv, b2d, wn)


def _tc_last(agg, hp, dinv, b2d, wf, bf2d):
    d = b2d.shape[1]
    dh = d // 2
    dout = wf.shape[1]

    def body(a_ref, hp_ref, dinv_ref, b_ref, w_ref, bf_ref, out_ref):
        z = jnp.concatenate([a_ref[0] + hp_ref[0], a_ref[1] + hp_ref[1]],
                            axis=1)
        dinv = dinv_ref[...]
        act = jnp.maximum(z * dinv + b_ref[...], 0.0)
        out_ref[...] = jnp.dot(act, w_ref[...],
                               preferred_element_type=jnp.float32) + bf_ref[...]

    return pl.pallas_call(
        body,
        grid=(NP // RB,),
        in_specs=[
            pl.BlockSpec((NC, RB, dh), lambda i: (0, i, 0)),
            pl.BlockSpec((NC, RB, dh), lambda i: (0, i, 0)),
            pl.BlockSpec((RB, 1), lambda i: (i, 0)),
            pl.BlockSpec((1, d), lambda i: (0, 0)),
            pl.BlockSpec(wf.shape, lambda i: (0, 0)),
            pl.BlockSpec((1, dout), lambda i: (0, 0)),
        ],
        out_specs=pl.BlockSpec((RB, dout), lambda i: (i, 0)),
        out_shape=jax.ShapeDtypeStruct((NP, dout), jnp.float32),
    )(agg, hp, dinv, b2d, wf, bf2d)


def kernel(x, edge_index, edge_attr, W1, b1, W2, b2, W3, b3, W4, b4, Wf, bf):
    pad = EP - E
    src_r = jnp.concatenate(
        [edge_index[0], jnp.zeros((pad,), edge_index.dtype)]
    ).reshape(NS, CH, C).astype(jnp.int32)
    dst_r = jnp.concatenate(
        [edge_index[1], jnp.zeros((pad,), edge_index.dtype)]
    ).reshape(NS, CH, C).astype(jnp.int32)
    ew_r = jnp.concatenate(
        [edge_attr, jnp.zeros((pad,), edge_attr.dtype)]
    ).reshape(NS, CH, C)
    xp = jnp.concatenate(
        [x, jnp.zeros((NP - N, x.shape[1]), x.dtype)], axis=0)

    deg_parts = _deg_kernel()(dst_r, ew_r, jnp.zeros((NP,), jnp.float32))
    degT = deg_parts.T

    dinv, hp = _tc_first(degT, xp, W1)

    for b_l, w_next in ((b1, W2), (b2, W3), (b3, W4)):
        dh = hp.shape[2]
        agg = _agg_kernel(dh)(hp, src_r, dst_r, ew_r,
                              jnp.zeros((NP, dh), jnp.float32))
        hp = _tc_mid(agg, hp, dinv, b_l.reshape(1, -1), w_next)

    dh = hp.shape[2]
    agg = _agg_kernel(dh)(hp, src_r, dst_r, ew_r,
                          jnp.zeros((NP, dh), jnp.float32))
    out = _tc_last(agg, hp, dinv, b4.reshape(1, -1), Wf, bf.reshape(1, -1))
    return out[:N]


# speedup vs baseline: 6.5464x; 6.5464x over previous
import functools

import jax
import jax.numpy as jnp
from jax import lax
from jax.experimental import pallas as pl
from jax.experimental.pallas import tpu as pltpu
from jax.experimental.pallas import tpu_sc as plsc

N = 10000
NP = 10240
E = 320000
NC, NS, LANES = 2, 16, 16
C = 128
CH = 160
EPT = CH * C
EP = NS * EPT
RB = 1280


def _sc_mesh():
    return plsc.VectorSubcoreMesh(core_axis_name="c", subcore_axis_name="s")


def _splat(vec, lane):
    idx = jnp.full((LANES, 1), lane, jnp.int32)
    dnums = lax.GatherDimensionNumbers(
        offset_dims=(), collapsed_slice_dims=(0,), start_index_map=(0,))
    return lax.gather(vec, idx, dnums, (1,),
                      mode=lax.GatherScatterMode.PROMISE_IN_BOUNDS)


@functools.lru_cache(maxsize=None)
def _deg_kernel():
    chc = CH // NC
    eptc = chc * C

    @functools.partial(
        pl.kernel,
        out_type=jax.ShapeDtypeStruct((NC, NP), jnp.float32),
        mesh=_sc_mesh(),
        scratch_types=[
            pltpu.VMEM((chc, C), jnp.int32),
            pltpu.VMEM((eptc,), jnp.float32),
            pltpu.VMEM_SHARED((NP,), jnp.float32),
        ],
    )
    def deg(dst_hbm, ew_hbm, zeros_hbm, out_hbm, dst_v, ew_v, acc):
        c = lax.axis_index("c")
        s = lax.axis_index("s")
        rps = NP // NS
        pltpu.sync_copy(zeros_hbm.at[pl.ds(s * rps, rps)],
                        acc.at[pl.ds(s * rps, rps)])
        pltpu.sync_copy(dst_hbm.at[s, pl.ds(c * chc, chc)], dst_v)
        pltpu.sync_copy(ew_hbm.at[s, pl.ds(c * eptc, eptc)], ew_v)
        plsc.subcore_barrier()

        @pl.loop(0, chc)
        def _(j):
            pltpu.sync_copy(ew_v.at[pl.ds(j * C, C)],
                            acc.at[dst_v.at[j]], add=True)

        plsc.subcore_barrier()

        @pl.when(s == 0)
        def _():
            pltpu.sync_copy(acc, out_hbm.at[c])

    return deg


@functools.lru_cache(maxsize=None)
def _agg_kernel(dh):
    @functools.partial(
        pl.kernel,
        out_type=jax.ShapeDtypeStruct((NC, NP, dh), jnp.float32),
        mesh=_sc_mesh(),
        compiler_params=pltpu.CompilerParams(use_tc_tiling_on_sc=False),
        scratch_types=[
            pltpu.VMEM((CH, C), jnp.int32),
            pltpu.VMEM((CH, C), jnp.int32),
            pltpu.VMEM((EPT,), jnp.float32),
            pltpu.VMEM((C, dh), jnp.float32),
            pltpu.VMEM_SHARED((NP, dh), jnp.float32),
            pltpu.SemaphoreType.DMA,
        ],
    )
    def agg(hp_hbm, src_hbm, dst_hbm, ew_hbm, zeros_hbm, out_hbm,
            src_v, dst_v, ew_v, rows_v, acc, sem):
        c = lax.axis_index("c")
        s = lax.axis_index("s")
        rps = NP // NS
        pltpu.sync_copy(zeros_hbm.at[pl.ds(s * rps, rps)],
                        acc.at[pl.ds(s * rps, rps)])
        pltpu.sync_copy(src_hbm.at[s], src_v)
        pltpu.sync_copy(dst_hbm.at[s], dst_v)
        pltpu.sync_copy(ew_hbm.at[s], ew_v)
        plsc.subcore_barrier()

        @pl.loop(0, CH)
        def _(j):
            pltpu.async_copy(hp_hbm.at[c].at[src_v.at[j]], rows_v, sem).wait()

            @pl.loop(0, C // LANES)
            def _(g):
                ewv = ew_v[pl.ds(j * C + g * LANES, LANES)]
                for r16 in range(LANES):
                    coef = _splat(ewv, r16)
                    r = g * LANES + r16
                    for k in range(dh // LANES):
                        sl = pl.ds(k * LANES, LANES)
                        rows_v[r, sl] = rows_v[r, sl] * coef

            pltpu.sync_copy(rows_v, acc.at[dst_v.at[j]], add=True)

        plsc.subcore_barrier()

        @pl.when(s == 0)
        def _():
            pltpu.sync_copy(acc, out_hbm.at[c])

    return agg


def _tc_first(degT, x, w1):
    d = w1.shape[1]
    dh = d // 2

    def body(deg_ref, x_ref, w_ref, dinv_ref, hp_ref):
        deg = deg_ref[:, 0] + deg_ref[:, 1] + 1.0
        dinv = jnp.where(deg > 0, lax.rsqrt(deg), 0.0)[:, None]
        dinv_ref[...] = dinv
        h = jnp.dot(x_ref[...], w_ref[...],
                    preferred_element_type=jnp.float32) * dinv
        hp_ref[0] = h[:, :dh]
        hp_ref[1] = h[:, dh:]

    return pl.pallas_call(
        body,
        grid=(NP // RB,),
        in_specs=[
            pl.BlockSpec((RB, 2), lambda i: (i, 0)),
            pl.BlockSpec((RB, x.shape[1]), lambda i: (i, 0)),
            pl.BlockSpec(w1.shape, lambda i: (0, 0)),
        ],
        out_specs=[
            pl.BlockSpec((RB, 1), lambda i: (i, 0)),
            pl.BlockSpec((NC, RB, dh), lambda i: (0, i, 0)),
        ],
        out_shape=[
            jax.ShapeDtypeStruct((NP, 1), jnp.float32),
            jax.ShapeDtypeStruct((NC, NP, dh), jnp.float32),
        ],
    )(degT, x, w1)


def _tc_mid(agg, hp, dinv, b2d, wn):
    d = b2d.shape[1]
    dh = d // 2
    dn = wn.shape[1]
    dnh = dn // 2

    def body(a_ref, hp_ref, dinv_ref, b_ref, w_ref, out_ref):
        z = jnp.concatenate([a_ref[0] + hp_ref[0], a_ref[1] + hp_ref[1]],
                            axis=1)
        dinv = dinv_ref[...]
        act = jnp.maximum(z * dinv + b_ref[...], 0.0)
        hn = jnp.dot(act, w_ref[...],
                     preferred_element_type=jnp.float32) * dinv
        out_ref[0] = hn[:, :dnh]
        out_ref[1] = hn[:, dnh:]

    return pl.pallas_call(
        body,
        grid=(NP // RB,),
        in_specs=[
            pl.BlockSpec((NC, RB, dh), lambda i: (0, i, 0)),
            pl.BlockSpec((NC, RB, dh), lambda i: (0, i, 0)),
            pl.BlockSpec((RB, 1), lambda i: (i, 0)),
            pl.BlockSpec((1, d), lambda i: (0, 0)),
            pl.BlockSpec(wn.shape, lambda i: (0, 0)),
        ],
        out_specs=pl.BlockSpec((NC, RB, dnh), lambda i: (0, i, 0)),
        out_shape=jax.ShapeDtypeStruct((NC, NP, dnh), jnp.float32),
    )(agg, hp, dinv, b2d, wn)


def _tc_last(agg, hp, dinv, b2d, wf, bf2d):
    d = b2d.shape[1]
    dh = d // 2
    dout = wf.shape[1]

    def body(a_ref, hp_ref, dinv_ref, b_ref, w_ref, bf_ref, out_ref):
        z = jnp.concatenate([a_ref[0] + hp_ref[0], a_ref[1] + hp_ref[1]],
                            axis=1)
        dinv = dinv_ref[...]
        act = jnp.maximum(z * dinv + b_ref[...], 0.0)
        out_ref[...] = jnp.dot(act, w_ref[...],
                               preferred_element_type=jnp.float32) + bf_ref[...]

    return pl.pallas_call(
        body,
        grid=(NP // RB,),
        in_specs=[
            pl.BlockSpec((NC, RB, dh), lambda i: (0, i, 0)),
            pl.BlockSpec((NC, RB, dh), lambda i: (0, i, 0)),
            pl.BlockSpec((RB, 1), lambda i: (i, 0)),
            pl.BlockSpec((1, d), lambda i: (0, 0)),
            pl.BlockSpec(wf.shape, lambda i: (0, 0)),
            pl.BlockSpec((1, dout), lambda i: (0, 0)),
        ],
        out_specs=pl.BlockSpec((RB, dout), lambda i: (i, 0)),
        out_shape=jax.ShapeDtypeStruct((NP, dout), jnp.float32),
    )(agg, hp, dinv, b2d, wf, bf2d)


def kernel(x, edge_index, edge_attr, W1, b1, W2, b2, W3, b3, W4, b4, Wf, bf):
    pad = EP - E
    src_r = jnp.concatenate(
        [edge_index[0], jnp.zeros((pad,), edge_index.dtype)]
    ).reshape(NS, CH, C).astype(jnp.int32)
    dst_r = jnp.concatenate(
        [edge_index[1], jnp.zeros((pad,), edge_index.dtype)]
    ).reshape(NS, CH, C).astype(jnp.int32)
    ew_r = jnp.concatenate(
        [edge_attr, jnp.zeros((pad,), edge_attr.dtype)]
    ).reshape(NS, EPT)
    xp = jnp.concatenate(
        [x, jnp.zeros((NP - N, x.shape[1]), x.dtype)], axis=0)

    deg_parts = _deg_kernel()(dst_r, ew_r, jnp.zeros((NP,), jnp.float32))
    degT = deg_parts.T

    dinv, hp = _tc_first(degT, xp, W1)

    for b_l, w_next in ((b1, W2), (b2, W3), (b3, W4)):
        dh = hp.shape[2]
        agg = _agg_kernel(dh)(hp, src_r, dst_r, ew_r,
                              jnp.zeros((NP, dh), jnp.float32))
        hp = _tc_mid(agg, hp, dinv, b_l.reshape(1, -1), w_next)

    dh = hp.shape[2]
    agg = _agg_kernel(dh)(hp, src_r, dst_r, ew_r,
                          jnp.zeros((NP, dh), jnp.float32))
    out = _tc_last(agg, hp, dinv, b4.reshape(1, -1), Wf, bf.reshape(1, -1))
    return out[:N]


# speedup vs baseline: 9.6492x; 1.4740x over previous
import functools

import jax
import jax.numpy as jnp
from jax import lax
from jax.experimental import pallas as pl
from jax.experimental.pallas import tpu as pltpu
from jax.experimental.pallas import tpu_sc as plsc

N = 10000
NP = 10240
E = 320000
NC, NS, LANES = 2, 16, 16
C = 128
CH = 160
EPT = CH * C
EP = NS * EPT
RB = 1280


def _sc_mesh():
    return plsc.VectorSubcoreMesh(core_axis_name="c", subcore_axis_name="s")


def _splat(vec, lane):
    idx = jnp.full((LANES, 1), lane, jnp.int32)
    dnums = lax.GatherDimensionNumbers(
        offset_dims=(), collapsed_slice_dims=(0,), start_index_map=(0,))
    return lax.gather(vec, idx, dnums, (1,),
                      mode=lax.GatherScatterMode.PROMISE_IN_BOUNDS)


@functools.lru_cache(maxsize=None)
def _deg_kernel():
    chc = CH // NC
    eptc = chc * C

    @functools.partial(
        pl.kernel,
        out_type=jax.ShapeDtypeStruct((NC, NP), jnp.float32),
        mesh=_sc_mesh(),
        scratch_types=[
            pltpu.VMEM((chc, C), jnp.int32),
            pltpu.VMEM((eptc,), jnp.float32),
            pltpu.VMEM_SHARED((NP,), jnp.float32),
        ],
    )
    def deg(dst_hbm, ew_hbm, zeros_hbm, out_hbm, dst_v, ew_v, acc):
        c = lax.axis_index("c")
        s = lax.axis_index("s")
        rps = NP // NS
        pltpu.sync_copy(zeros_hbm.at[pl.ds(s * rps, rps)],
                        acc.at[pl.ds(s * rps, rps)])
        pltpu.sync_copy(dst_hbm.at[s, pl.ds(c * chc, chc)], dst_v)
        pltpu.sync_copy(ew_hbm.at[s, pl.ds(c * eptc, eptc)], ew_v)
        plsc.subcore_barrier()

        @pl.loop(0, chc)
        def _(j):
            pltpu.sync_copy(ew_v.at[pl.ds(j * C, C)],
                            acc.at[dst_v.at[j]], add=True)

        plsc.subcore_barrier()

        @pl.when(s == 0)
        def _():
            pltpu.sync_copy(acc, out_hbm.at[c])

    return deg


@functools.lru_cache(maxsize=None)
def _agg_kernel(dh):
    @functools.partial(
        pl.kernel,
        out_type=jax.ShapeDtypeStruct((NC, NP, dh), jnp.float32),
        mesh=_sc_mesh(),
        compiler_params=pltpu.CompilerParams(use_tc_tiling_on_sc=False),
        scratch_types=[
            pltpu.VMEM((CH, C), jnp.int32),
            pltpu.VMEM((CH, C), jnp.int32),
            pltpu.VMEM((EPT,), jnp.float32),
            pltpu.VMEM((C, dh), jnp.float32),
            pltpu.VMEM((C, dh), jnp.float32),
            pltpu.VMEM_SHARED((NP, dh), jnp.float32),
            pltpu.SemaphoreType.DMA,
            pltpu.SemaphoreType.DMA,
        ],
    )
    def agg(hp_hbm, src_hbm, dst_hbm, ew_hbm, zeros_hbm, out_hbm,
            src_v, dst_v, ew_v, rows0, rows1, acc, sem0, sem1):
        c = lax.axis_index("c")
        s = lax.axis_index("s")
        rps = NP // NS
        pltpu.sync_copy(zeros_hbm.at[pl.ds(s * rps, rps)],
                        acc.at[pl.ds(s * rps, rps)])
        pltpu.sync_copy(src_hbm.at[s], src_v)
        pltpu.sync_copy(dst_hbm.at[s], dst_v)
        pltpu.sync_copy(ew_hbm.at[s], ew_v)
        plsc.subcore_barrier()

        def scale(rows_v, j):
            @pl.loop(0, C // LANES)
            def _(g):
                ewv = ew_v[pl.ds(j * C + g * LANES, LANES)]
                for r16 in range(LANES):
                    coef = _splat(ewv, r16)
                    r = g * LANES + r16
                    for k in range(dh // LANES):
                        sl = pl.ds(k * LANES, LANES)
                        rows_v[r, sl] = rows_v[r, sl] * coef

        HF = CH // 2
        pltpu.async_copy(hp_hbm.at[c].at[src_v.at[0]], rows0, sem0)

        @pl.loop(0, HF)
        def _(jj):
            j0 = jj * 2
            j1 = j0 + 1
            pltpu.async_copy(hp_hbm.at[c].at[src_v.at[j1]], rows1, sem1)
            pltpu.make_async_copy(hp_hbm.at[c].at[src_v.at[j0]],
                                  rows0, sem0).wait()
            scale(rows0, j0)
            pltpu.sync_copy(rows0, acc.at[dst_v.at[j0]], add=True)

            @pl.when(jj < HF - 1)
            def _():
                pltpu.async_copy(hp_hbm.at[c].at[src_v.at[j0 + 2]],
                                 rows0, sem0)

            pltpu.make_async_copy(hp_hbm.at[c].at[src_v.at[j1]],
                                  rows1, sem1).wait()
            scale(rows1, j1)
            pltpu.sync_copy(rows1, acc.at[dst_v.at[j1]], add=True)

        plsc.subcore_barrier()

        @pl.when(s == 0)
        def _():
            pltpu.sync_copy(acc, out_hbm.at[c])

    return agg


def _tc_first(degT, x, w1):
    d = w1.shape[1]
    dh = d // 2

    def body(deg_ref, x_ref, w_ref, dinv_ref, hp_ref):
        deg = deg_ref[:, 0] + deg_ref[:, 1] + 1.0
        dinv = jnp.where(deg > 0, lax.rsqrt(deg), 0.0)[:, None]
        dinv_ref[...] = dinv
        h = jnp.dot(x_ref[...], w_ref[...],
                    preferred_element_type=jnp.float32) * dinv
        hp_ref[0] = h[:, :dh]
        hp_ref[1] = h[:, dh:]

    return pl.pallas_call(
        body,
        grid=(NP // RB,),
        in_specs=[
            pl.BlockSpec((RB, 2), lambda i: (i, 0)),
            pl.BlockSpec((RB, x.shape[1]), lambda i: (i, 0)),
            pl.BlockSpec(w1.shape, lambda i: (0, 0)),
        ],
        out_specs=[
            pl.BlockSpec((RB, 1), lambda i: (i, 0)),
            pl.BlockSpec((NC, RB, dh), lambda i: (0, i, 0)),
        ],
        out_shape=[
            jax.ShapeDtypeStruct((NP, 1), jnp.float32),
            jax.ShapeDtypeStruct((NC, NP, dh), jnp.float32),
        ],
    )(degT, x, w1)


def _tc_mid(agg, hp, dinv, b2d, wn):
    d = b2d.shape[1]
    dh = d // 2
    dn = wn.shape[1]
    dnh = dn // 2

    def body(a_ref, hp_ref, dinv_ref, b_ref, w_ref, out_ref):
        z = jnp.concatenate([a_ref[0] + hp_ref[0], a_ref[1] + hp_ref[1]],
                            axis=1)
        dinv = dinv_ref[...]
        act = jnp.maximum(z * dinv + b_ref[...], 0.0)
        hn = jnp.dot(act, w_ref[...],
                     preferred_element_type=jnp.float32) * dinv
        out_ref[0] = hn[:, :dnh]
        out_ref[1] = hn[:, dnh:]

    return pl.pallas_call(
        body,
        grid=(NP // RB,),
        in_specs=[
            pl.BlockSpec((NC, RB, dh), lambda i: (0, i, 0)),
            pl.BlockSpec((NC, RB, dh), lambda i: (0, i, 0)),
            pl.BlockSpec((RB, 1), lambda i: (i, 0)),
            pl.BlockSpec((1, d), lambda i: (0, 0)),
            pl.BlockSpec(wn.shape, lambda i: (0, 0)),
        ],
        out_specs=pl.BlockSpec((NC, RB, dnh), lambda i: (0, i, 0)),
        out_shape=jax.ShapeDtypeStruct((NC, NP, dnh), jnp.float32),
    )(agg, hp, dinv, b2d, wn)


def _tc_last(agg, hp, dinv, b2d, wf, bf2d):
    d = b2d.shape[1]
    dh = d // 2
    dout = wf.shape[1]

    def body(a_ref, hp_ref, dinv_ref, b_ref, w_ref, bf_ref, out_ref):
        z = jnp.concatenate([a_ref[0] + hp_ref[0], a_ref[1] + hp_ref[1]],
                            axis=1)
        dinv = dinv_ref[...]
        act = jnp.maximum(z * dinv + b_ref[...], 0.0)
        out_ref[...] = jnp.dot(act, w_ref[...],
                               preferred_element_type=jnp.float32) + bf_ref[...]

    return pl.pallas_call(
        body,
        grid=(NP // RB,),
        in_specs=[
            pl.BlockSpec((NC, RB, dh), lambda i: (0, i, 0)),
            pl.BlockSpec((NC, RB, dh), lambda i: (0, i, 0)),
            pl.BlockSpec((RB, 1), lambda i: (i, 0)),
            pl.BlockSpec((1, d), lambda i: (0, 0)),
            pl.BlockSpec(wf.shape, lambda i: (0, 0)),
            pl.BlockSpec((1, dout), lambda i: (0, 0)),
        ],
        out_specs=pl.BlockSpec((RB, dout), lambda i: (i, 0)),
        out_shape=jax.ShapeDtypeStruct((NP, dout), jnp.float32),
    )(agg, hp, dinv, b2d, wf, bf2d)


def kernel(x, edge_index, edge_attr, W1, b1, W2, b2, W3, b3, W4, b4, Wf, bf):
    pad = EP - E
    src_r = jnp.concatenate(
        [edge_index[0], jnp.zeros((pad,), edge_index.dtype)]
    ).reshape(NS, CH, C).astype(jnp.int32)
    dst_r = jnp.concatenate(
        [edge_index[1], jnp.zeros((pad,), edge_index.dtype)]
    ).reshape(NS, CH, C).astype(jnp.int32)
    ew_r = jnp.concatenate(
        [edge_attr, jnp.zeros((pad,), edge_attr.dtype)]
    ).reshape(NS, EPT)
    xp = jnp.concatenate(
        [x, jnp.zeros((NP - N, x.shape[1]), x.dtype)], axis=0)

    deg_parts = _deg_kernel()(dst_r, ew_r, jnp.zeros((NP,), jnp.float32))
    degT = deg_parts.T

    dinv, hp = _tc_first(degT, xp, W1)

    for b_l, w_next in ((b1, W2), (b2, W3), (b3, W4)):
        dh = hp.shape[2]
        agg = _agg_kernel(dh)(hp, src_r, dst_r, ew_r,
                              jnp.zeros((NP, dh), jnp.float32))
        hp = _tc_mid(agg, hp, dinv, b_l.reshape(1, -1), w_next)

    dh = hp.shape[2]
    agg = _agg_kernel(dh)(hp, src_r, dst_r, ew_r,
                          jnp.zeros((NP, dh), jnp.float32))
    out = _tc_last(agg, hp, dinv, b4.reshape(1, -1), Wf, bf.reshape(1, -1))
    return out[:N]


# speedup vs baseline: 10.4195x; 1.0798x over previous
import functools

import jax
import jax.numpy as jnp
from jax import lax
from jax.experimental import pallas as pl
from jax.experimental.pallas import tpu as pltpu
from jax.experimental.pallas import tpu_sc as plsc

N = 10000
NP = 10240
E = 320000
NC, NS, LANES = 2, 16, 16
C = 128
CH = 160
EPT = CH * C
EP = NS * EPT
RB = 1280


def _sc_mesh():
    return plsc.VectorSubcoreMesh(core_axis_name="c", subcore_axis_name="s")


def _splat(vec, lane):
    idx = jnp.full((LANES, 1), lane, jnp.int32)
    dnums = lax.GatherDimensionNumbers(
        offset_dims=(), collapsed_slice_dims=(0,), start_index_map=(0,))
    return lax.gather(vec, idx, dnums, (1,),
                      mode=lax.GatherScatterMode.PROMISE_IN_BOUNDS)


@functools.lru_cache(maxsize=None)
def _deg_kernel():
    chc = CH // NC
    eptc = chc * C

    @functools.partial(
        pl.kernel,
        out_type=jax.ShapeDtypeStruct((NC, NP), jnp.float32),
        mesh=_sc_mesh(),
        scratch_types=[
            pltpu.VMEM((chc, C), jnp.int32),
            pltpu.VMEM((eptc,), jnp.float32),
            pltpu.VMEM_SHARED((NP,), jnp.float32),
        ],
    )
    def deg(dst_hbm, ew_hbm, zeros_hbm, out_hbm, dst_v, ew_v, acc):
        c = lax.axis_index("c")
        s = lax.axis_index("s")
        rps = NP // NS
        pltpu.sync_copy(zeros_hbm.at[pl.ds(s * rps, rps)],
                        acc.at[pl.ds(s * rps, rps)])
        pltpu.sync_copy(dst_hbm.at[s, pl.ds(c * chc, chc)], dst_v)
        pltpu.sync_copy(ew_hbm.at[s, pl.ds(c * eptc, eptc)], ew_v)
        plsc.subcore_barrier()

        @pl.loop(0, chc)
        def _(j):
            pltpu.sync_copy(ew_v.at[pl.ds(j * C, C)],
                            acc.at[dst_v.at[j]], add=True)

        plsc.subcore_barrier()

        @pl.when(s == 0)
        def _():
            pltpu.sync_copy(acc, out_hbm.at[c])

    return deg


@functools.lru_cache(maxsize=None)
def _agg_kernel(dh):
    @functools.partial(
        pl.kernel,
        out_type=jax.ShapeDtypeStruct((NC, NP, dh), jnp.float32),
        mesh=_sc_mesh(),
        compiler_params=pltpu.CompilerParams(use_tc_tiling_on_sc=False),
        scratch_types=[
            pltpu.VMEM((CH // 2, C), jnp.int32),
            pltpu.VMEM((CH // 2, C), jnp.int32),
            pltpu.VMEM((EPT // 2,), jnp.float32),
            pltpu.VMEM((C, dh), jnp.float32),
            pltpu.VMEM((C, dh), jnp.float32),
            pltpu.VMEM((C, dh), jnp.float32),
            pltpu.VMEM((C, dh), jnp.float32),
            pltpu.VMEM_SHARED((NP, dh), jnp.float32),
            pltpu.SemaphoreType.DMA,
            pltpu.SemaphoreType.DMA,
            pltpu.SemaphoreType.DMA,
            pltpu.SemaphoreType.DMA,
            pltpu.SemaphoreType.DMA,
            pltpu.SemaphoreType.DMA,
            pltpu.SemaphoreType.DMA,
            pltpu.SemaphoreType.DMA,
        ],
    )
    def agg(hp_hbm, src_hbm, dst_hbm, ew_hbm, zeros_hbm, out_hbm,
            src_v, dst_v, ew_v, r0, r1, r2, r3, acc,
            g0, g1, g2, g3, s0, s1, s2, s3):
        rows = (r0, r1, r2, r3)
        gsem = (g0, g1, g2, g3)
        ssem = (s0, s1, s2, s3)
        c = lax.axis_index("c")
        s = lax.axis_index("s")
        rps = NP // NS
        pltpu.sync_copy(zeros_hbm.at[pl.ds(s * rps, rps)],
                        acc.at[pl.ds(s * rps, rps)])
        plsc.subcore_barrier()

        def scale(rows_v, j):
            @pl.loop(0, C // LANES)
            def _(g):
                ewv = ew_v[pl.ds(j * C + g * LANES, LANES)]
                for r16 in range(LANES):
                    coef = _splat(ewv, r16)
                    r = g * LANES + r16
                    for k in range(dh // LANES):
                        sl = pl.ds(k * LANES, LANES)
                        rows_v[r, sl] = rows_v[r, sl] * coef

        def drain(sem, buf):
            pltpu.make_async_copy(zeros_hbm.at[pl.ds(0, C)], buf, sem).wait()

        CH2 = CH // 2
        HF4 = CH2 // 4
        for p in range(2):
            pltpu.sync_copy(src_hbm.at[s, pl.ds(p * CH2, CH2)], src_v)
            pltpu.sync_copy(dst_hbm.at[s, pl.ds(p * CH2, CH2)], dst_v)
            pltpu.sync_copy(ew_hbm.at[s, pl.ds(p * CH2 * C, CH2 * C)], ew_v)

            pltpu.async_copy(hp_hbm.at[c].at[src_v.at[0]], rows[0], gsem[0])
            pltpu.async_copy(hp_hbm.at[c].at[src_v.at[1]], rows[1], gsem[1])

            @pl.loop(0, HF4)
            def _(jj):
                for b in range(4):
                    j = jj * 4 + b
                    b2 = (b + 2) % 4
                    jn = j + 2
                    if b < 2:
                        @pl.when(jj > 0)
                        def _():
                            drain(ssem[b2], rows[b2])
                        pltpu.async_copy(hp_hbm.at[c].at[src_v.at[jn]],
                                         rows[b2], gsem[b2])
                    else:
                        drain(ssem[b2], rows[b2])

                        @pl.when(jj < HF4 - 1)
                        def _():
                            pltpu.async_copy(hp_hbm.at[c].at[src_v.at[jn]],
                                             rows[b2], gsem[b2])

                    drain(gsem[b], rows[b])
                    scale(rows[b], j)
                    pltpu.async_copy(rows[b], acc.at[dst_v.at[j]], ssem[b],
                                     add=True)

            drain(ssem[2], rows[2])
            drain(ssem[3], rows[3])

        plsc.subcore_barrier()

        @pl.when(s == 0)
        def _():
            pltpu.sync_copy(acc, out_hbm.at[c])

    return agg


def _tc_first(degT, x, w1):
    d = w1.shape[1]
    dh = d // 2

    def body(deg_ref, x_ref, w_ref, dinv_ref, hp_ref):
        deg = deg_ref[:, 0] + deg_ref[:, 1] + 1.0
        dinv = jnp.where(deg > 0, lax.rsqrt(deg), 0.0)[:, None]
        dinv_ref[...] = dinv
        h = jnp.dot(x_ref[...], w_ref[...],
                    preferred_element_type=jnp.float32) * dinv
        hp_ref[0] = h[:, :dh]
        hp_ref[1] = h[:, dh:]

    return pl.pallas_call(
        body,
        grid=(NP // RB,),
        in_specs=[
            pl.BlockSpec((RB, 2), lambda i: (i, 0)),
            pl.BlockSpec((RB, x.shape[1]), lambda i: (i, 0)),
            pl.BlockSpec(w1.shape, lambda i: (0, 0)),
        ],
        out_specs=[
            pl.BlockSpec((RB, 1), lambda i: (i, 0)),
            pl.BlockSpec((NC, RB, dh), lambda i: (0, i, 0)),
        ],
        out_shape=[
            jax.ShapeDtypeStruct((NP, 1), jnp.float32),
            jax.ShapeDtypeStruct((NC, NP, dh), jnp.float32),
        ],
    )(degT, x, w1)


def _tc_mid(agg, hp, dinv, b2d, wn):
    d = b2d.shape[1]
    dh = d // 2
    dn = wn.shape[1]
    dnh = dn // 2

    def body(a_ref, hp_ref, dinv_ref, b_ref, w_ref, out_ref):
        z = jnp.concatenate([a_ref[0] + hp_ref[0], a_ref[1] + hp_ref[1]],
                            axis=1)
        dinv = dinv_ref[...]
        act = jnp.maximum(z * dinv + b_ref[...], 0.0)
        hn = jnp.dot(act, w_ref[...],
                     preferred_element_type=jnp.float32) * dinv
        out_ref[0] = hn[:, :dnh]
        out_ref[1] = hn[:, dnh:]

    return pl.pallas_call(
        body,
        grid=(NP // RB,),
        in_specs=[
            pl.BlockSpec((NC, RB, dh), lambda i: (0, i, 0)),
            pl.BlockSpec((NC, RB, dh), lambda i: (0, i, 0)),
            pl.BlockSpec((RB, 1), lambda i: (i, 0)),
            pl.BlockSpec((1, d), lambda i: (0, 0)),
            pl.BlockSpec(wn.shape, lambda i: (0, 0)),
        ],
        out_specs=pl.BlockSpec((NC, RB, dnh), lambda i: (0, i, 0)),
        out_shape=jax.ShapeDtypeStruct((NC, NP, dnh), jnp.float32),
    )(agg, hp, dinv, b2d, wn)


def _tc_last(agg, hp, dinv, b2d, wf, bf2d):
    d = b2d.shape[1]
    dh = d // 2
    dout = wf.shape[1]

    def body(a_ref, hp_ref, dinv_ref, b_ref, w_ref, bf_ref, out_ref):
        z = jnp.concatenate([a_ref[0] + hp_ref[0], a_ref[1] + hp_ref[1]],
                            axis=1)
        dinv = dinv_ref[...]
        act = jnp.maximum(z * dinv + b_ref[...], 0.0)
        out_ref[...] = jnp.dot(act, w_ref[...],
                               preferred_element_type=jnp.float32) + bf_ref[...]

    return pl.pallas_call(
        body,
        grid=(NP // RB,),
        in_specs=[
            pl.BlockSpec((NC, RB, dh), lambda i: (0, i, 0)),
            pl.BlockSpec((NC, RB, dh), lambda i: (0, i, 0)),
            pl.BlockSpec((RB, 1), lambda i: (i, 0)),
            pl.BlockSpec((1, d), lambda i: (0, 0)),
            pl.BlockSpec(wf.shape, lambda i: (0, 0)),
            pl.BlockSpec((1, dout), lambda i: (0, 0)),
        ],
        out_specs=pl.BlockSpec((RB, dout), lambda i: (i, 0)),
        out_shape=jax.ShapeDtypeStruct((NP, dout), jnp.float32),
    )(agg, hp, dinv, b2d, wf, bf2d)


def kernel(x, edge_index, edge_attr, W1, b1, W2, b2, W3, b3, W4, b4, Wf, bf):
    pad = EP - E
    src_r = jnp.concatenate(
        [edge_index[0], jnp.zeros((pad,), edge_index.dtype)]
    ).reshape(NS, CH, C).astype(jnp.int32)
    dst_r = jnp.concatenate(
        [edge_index[1], jnp.zeros((pad,), edge_index.dtype)]
    ).reshape(NS, CH, C).astype(jnp.int32)
    ew_r = jnp.concatenate(
        [edge_attr, jnp.zeros((pad,), edge_attr.dtype)]
    ).reshape(NS, EPT)
    xp = jnp.concatenate(
        [x, jnp.zeros((NP - N, x.shape[1]), x.dtype)], axis=0)

    deg_parts = _deg_kernel()(dst_r, ew_r, jnp.zeros((NP,), jnp.float32))
    degT = deg_parts.T

    dinv, hp = _tc_first(degT, xp, W1)

    for b_l, w_next in ((b1, W2), (b2, W3), (b3, W4)):
        dh = hp.shape[2]
        agg = _agg_kernel(dh)(hp, src_r, dst_r, ew_r,
                              jnp.zeros((NP, dh), jnp.float32))
        hp = _tc_mid(agg, hp, dinv, b_l.reshape(1, -1), w_next)

    dh = hp.shape[2]
    agg = _agg_kernel(dh)(hp, src_r, dst_r, ew_r,
                          jnp.zeros((NP, dh), jnp.float32))
    out = _tc_last(agg, hp, dinv, b4.reshape(1, -1), Wf, bf.reshape(1, -1))
    return out[:N]


# speedup vs baseline: 13.6114x; 1.3063x over previous
import functools

import jax
import jax.numpy as jnp
from jax import lax
from jax.experimental import pallas as pl
from jax.experimental.pallas import tpu as pltpu
from jax.experimental.pallas import tpu_sc as plsc

N = 10000
NP = 10240
E = 320000
NC, NS, LANES = 2, 16, 16
C = 128
CH = 160
EPT = CH * C
EP = NS * EPT
RB = 1280


def _sc_mesh():
    return plsc.VectorSubcoreMesh(core_axis_name="c", subcore_axis_name="s")


def _splat(vec, lane):
    idx = jnp.full((LANES, 1), lane, jnp.int32)
    dnums = lax.GatherDimensionNumbers(
        offset_dims=(), collapsed_slice_dims=(0,), start_index_map=(0,))
    return lax.gather(vec, idx, dnums, (1,),
                      mode=lax.GatherScatterMode.PROMISE_IN_BOUNDS)


@functools.lru_cache(maxsize=None)
def _deg_kernel():
    chc = CH // NC
    eptc = chc * C

    @functools.partial(
        pl.kernel,
        out_type=jax.ShapeDtypeStruct((NC, NP), jnp.float32),
        mesh=_sc_mesh(),
        scratch_types=[
            pltpu.VMEM((chc, C), jnp.int32),
            pltpu.VMEM((eptc,), jnp.float32),
            pltpu.VMEM_SHARED((NP,), jnp.float32),
        ],
    )
    def deg(dst_hbm, ew_hbm, zeros_hbm, out_hbm, dst_v, ew_v, acc):
        c = lax.axis_index("c")
        s = lax.axis_index("s")
        rps = NP // NS
        pltpu.sync_copy(zeros_hbm.at[pl.ds(s * rps, rps)],
                        acc.at[pl.ds(s * rps, rps)])
        pltpu.sync_copy(dst_hbm.at[s, pl.ds(c * chc, chc)], dst_v)
        pltpu.sync_copy(ew_hbm.at[s, pl.ds(c * eptc, eptc)], ew_v)
        plsc.subcore_barrier()

        @pl.loop(0, chc)
        def _(j):
            pltpu.sync_copy(ew_v.at[pl.ds(j * C, C)],
                            acc.at[dst_v.at[j]], add=True)

        plsc.subcore_barrier()

        @pl.when(s == 0)
        def _():
            pltpu.sync_copy(acc, out_hbm.at[c])

    return deg


@functools.lru_cache(maxsize=None)
def _agg_kernel(dh):
    @functools.partial(
        pl.kernel,
        out_type=jax.ShapeDtypeStruct((NC, NP, dh), jnp.float32),
        mesh=_sc_mesh(),
        compiler_params=pltpu.CompilerParams(use_tc_tiling_on_sc=False),
        scratch_types=[
            pltpu.VMEM((CH // 2, C), jnp.int32),
            pltpu.VMEM((CH // 2, C), jnp.int32),
            pltpu.VMEM((EPT // 2,), jnp.float32),
            pltpu.VMEM((C, dh), jnp.float32),
            pltpu.VMEM((C, dh), jnp.float32),
            pltpu.VMEM((C, dh), jnp.float32),
            pltpu.VMEM((C, dh), jnp.float32),
            pltpu.VMEM_SHARED((NP, dh), jnp.float32),
            pltpu.SemaphoreType.DMA,
            pltpu.SemaphoreType.DMA,
            pltpu.SemaphoreType.DMA,
            pltpu.SemaphoreType.DMA,
            pltpu.SemaphoreType.DMA,
            pltpu.SemaphoreType.DMA,
            pltpu.SemaphoreType.DMA,
            pltpu.SemaphoreType.DMA,
        ],
    )
    def agg(hp_hbm, src_hbm, dst_hbm, ew_hbm, zeros_hbm, out_hbm,
            src_v, dst_v, ew_v, r0, r1, r2, r3, acc,
            g0, g1, g2, g3, s0, s1, s2, s3):
        rows = (r0, r1, r2, r3)
        gsem = (g0, g1, g2, g3)
        ssem = (s0, s1, s2, s3)
        c = lax.axis_index("c")
        s = lax.axis_index("s")
        rps = NP // NS
        pltpu.sync_copy(zeros_hbm.at[pl.ds(s * rps, rps)],
                        acc.at[pl.ds(s * rps, rps)])
        plsc.subcore_barrier()

        def scale(rows_v, j):
            @pl.loop(0, C // LANES)
            def _(g):
                ewv = ew_v[pl.ds(j * C + g * LANES, LANES)]
                for r16 in range(LANES):
                    coef = _splat(ewv, r16)
                    r = g * LANES + r16
                    for k in range(dh // LANES):
                        sl = pl.ds(k * LANES, LANES)
                        rows_v[r, sl] = rows_v[r, sl] * coef

        def drain(sem, buf):
            pltpu.make_async_copy(zeros_hbm.at[pl.ds(0, C)], buf, sem).wait()

        CH2 = CH // 2
        HF4 = CH2 // 4
        for p in range(2):
            pltpu.sync_copy(src_hbm.at[s, pl.ds(p * CH2, CH2)], src_v)
            pltpu.sync_copy(dst_hbm.at[s, pl.ds(p * CH2, CH2)], dst_v)
            pltpu.sync_copy(ew_hbm.at[s, pl.ds(p * CH2 * C, CH2 * C)], ew_v)

            pltpu.async_copy(hp_hbm.at[c].at[src_v.at[0]], rows[0], gsem[0])
            pltpu.async_copy(hp_hbm.at[c].at[src_v.at[1]], rows[1], gsem[1])

            @pl.loop(0, HF4)
            def _(jj):
                for b in range(4):
                    j = jj * 4 + b
                    b2 = (b + 2) % 4
                    jn = j + 2
                    if b < 2:
                        @pl.when(jj > 0)
                        def _():
                            drain(ssem[b2], rows[b2])
                        pltpu.async_copy(hp_hbm.at[c].at[src_v.at[jn]],
                                         rows[b2], gsem[b2])
                    else:
                        drain(ssem[b2], rows[b2])

                        @pl.when(jj < HF4 - 1)
                        def _():
                            pltpu.async_copy(hp_hbm.at[c].at[src_v.at[jn]],
                                             rows[b2], gsem[b2])

                    drain(gsem[b], rows[b])
                    scale(rows[b], j)
                    pltpu.async_copy(rows[b], acc.at[dst_v.at[j]], ssem[b],
                                     add=True)

            drain(ssem[2], rows[2])
            drain(ssem[3], rows[3])

        plsc.subcore_barrier()

        @pl.when(s == 0)
        def _():
            pltpu.sync_copy(acc, out_hbm.at[c])

    return agg


def _tc_first(degT, x, w1):
    d = w1.shape[1]
    dh = d // 2

    def body(deg_ref, x_ref, w_ref, dinv_ref, hp_ref):
        deg = deg_ref[:, 0] + deg_ref[:, 1] + 1.0
        dinv = jnp.where(deg > 0, lax.rsqrt(deg), 0.0)[:, None]
        dinv_ref[...] = dinv
        h = jnp.dot(x_ref[...], w_ref[...],
                    preferred_element_type=jnp.float32) * dinv
        hp_ref[0] = h[:, :dh]
        hp_ref[1] = h[:, dh:]

    return pl.pallas_call(
        body,
        grid=(NP // RB,),
        in_specs=[
            pl.BlockSpec((RB, 2), lambda i: (i, 0)),
            pl.BlockSpec((RB, x.shape[1]), lambda i: (i, 0)),
            pl.BlockSpec(w1.shape, lambda i: (0, 0)),
        ],
        out_specs=[
            pl.BlockSpec((RB, 1), lambda i: (i, 0)),
            pl.BlockSpec((NC, RB, dh), lambda i: (0, i, 0)),
        ],
        out_shape=[
            jax.ShapeDtypeStruct((NP, 1), jnp.float32),
            jax.ShapeDtypeStruct((NC, NP, dh), jnp.float32),
        ],
    )(degT, x, w1)


def _tc_mid(agg, hp, dinv, b2d, wn):
    d = b2d.shape[1]
    dh = d // 2
    dn = wn.shape[1]
    dnh = dn // 2

    def body(a_ref, hp_ref, dinv_ref, b_ref, w_ref, out_ref):
        z = jnp.concatenate([a_ref[0] + hp_ref[0], a_ref[1] + hp_ref[1]],
                            axis=1)
        dinv = dinv_ref[...]
        act = jnp.maximum(z * dinv + b_ref[...], 0.0)
        hn = jnp.dot(act, w_ref[...],
                     preferred_element_type=jnp.float32) * dinv
        out_ref[0] = hn[:, :dnh]
        out_ref[1] = hn[:, dnh:]

    return pl.pallas_call(
        body,
        grid=(NP // RB,),
        in_specs=[
            pl.BlockSpec((NC, RB, dh), lambda i: (0, i, 0)),
            pl.BlockSpec((NC, RB, dh), lambda i: (0, i, 0)),
            pl.BlockSpec((RB, 1), lambda i: (i, 0)),
            pl.BlockSpec((1, d), lambda i: (0, 0)),
            pl.BlockSpec(wn.shape, lambda i: (0, 0)),
        ],
        out_specs=pl.BlockSpec((NC, RB, dnh), lambda i: (0, i, 0)),
        out_shape=jax.ShapeDtypeStruct((NC, NP, dnh), jnp.float32),
    )(agg, hp, dinv, b2d, wn)


def _tc_last(agg, hp, dinv, b2d, wf, bf2d):
    d = b2d.shape[1]
    dh = d // 2
    dout = wf.shape[1]

    def body(a_ref, hp_ref, dinv_ref, b_ref, w_ref, bf_ref, out_ref):
        z = jnp.concatenate([a_ref[0] + hp_ref[0], a_ref[1] + hp_ref[1]],
                            axis=1)
        dinv = dinv_ref[...]
        act = jnp.maximum(z * dinv + b_ref[...], 0.0)
        out_ref[...] = jnp.dot(act, w_ref[...],
                               preferred_element_type=jnp.float32) + bf_ref[...]

    return pl.pallas_call(
        body,
        grid=(NP // RB,),
        in_specs=[
            pl.BlockSpec((NC, RB, dh), lambda i: (0, i, 0)),
            pl.BlockSpec((NC, RB, dh), lambda i: (0, i, 0)),
            pl.BlockSpec((RB, 1), lambda i: (i, 0)),
            pl.BlockSpec((1, d), lambda i: (0, 0)),
            pl.BlockSpec(wf.shape, lambda i: (0, 0)),
            pl.BlockSpec((1, dout), lambda i: (0, 0)),
        ],
        out_specs=pl.BlockSpec((RB, dout), lambda i: (i, 0)),
        out_shape=jax.ShapeDtypeStruct((NP, dout), jnp.float32),
    )(agg, hp, dinv, b2d, wf, bf2d)


def kernel(x, edge_index, edge_attr, W1, b1, W2, b2, W3, b3, W4, b4, Wf, bf):
    pad = EP - E
    pad_idx = jnp.arange(pad, dtype=edge_index.dtype) % N
    src_r = jnp.concatenate(
        [edge_index[0], pad_idx]
    ).reshape(NS, CH, C).astype(jnp.int32)
    dst_r = jnp.concatenate(
        [edge_index[1], pad_idx]
    ).reshape(NS, CH, C).astype(jnp.int32)
    ew_r = jnp.concatenate(
        [edge_attr, jnp.zeros((pad,), edge_attr.dtype)]
    ).reshape(NS, EPT)
    xp = jnp.concatenate(
        [x, jnp.zeros((NP - N, x.shape[1]), x.dtype)], axis=0)

    deg_parts = _deg_kernel()(dst_r, ew_r, jnp.zeros((NP,), jnp.float32))
    degT = deg_parts.T

    dinv, hp = _tc_first(degT, xp, W1)

    for b_l, w_next in ((b1, W2), (b2, W3), (b3, W4)):
        dh = hp.shape[2]
        agg = _agg_kernel(dh)(hp, src_r, dst_r, ew_r,
                              jnp.zeros((NP, dh), jnp.float32))
        hp = _tc_mid(agg, hp, dinv, b_l.reshape(1, -1), w_next)

    dh = hp.shape[2]
    agg = _agg_kernel(dh)(hp, src_r, dst_r, ew_r,
                          jnp.zeros((NP, dh), jnp.float32))
    out = _tc_last(agg, hp, dinv, b4.reshape(1, -1), Wf, bf.reshape(1, -1))
    return out[:N]


# speedup vs baseline: 20.5158x; 1.5072x over previous
import functools

import jax
import jax.numpy as jnp
from jax import lax
from jax.experimental import pallas as pl
from jax.experimental.pallas import tpu as pltpu
from jax.experimental.pallas import tpu_sc as plsc

N = 10000
NP = 10240
E = 320000
NC, NS, LANES = 2, 16, 16
C = 128
CH = 160
EPT = CH * C
EP = NS * EPT
RB = 1280


def _groups(d):
    return 2 if d <= 64 else 4


def _sc_mesh():
    return plsc.VectorSubcoreMesh(core_axis_name="c", subcore_axis_name="s")


def _splat(vec, lane):
    idx = jnp.full((LANES, 1), lane, jnp.int32)
    dnums = lax.GatherDimensionNumbers(
        offset_dims=(), collapsed_slice_dims=(0,), start_index_map=(0,))
    return lax.gather(vec, idx, dnums, (1,),
                      mode=lax.GatherScatterMode.PROMISE_IN_BOUNDS)


def _unpack(psd_v, src_v, dst_v, nch):
    @pl.loop(0, nch)
    def _(j):
        @pl.loop(0, C // LANES)
        def _(g):
            sl = pl.ds(g * LANES, LANES)
            v = psd_v[j, sl]
            if src_v is not None:
                src_v[j, sl] = lax.shift_right_logical(v, 16)
            dst_v[j, sl] = lax.bitwise_and(v, 0xFFFF)


@functools.lru_cache(maxsize=None)
def _deg_kernel():
    chc = CH // NC
    eptc = chc * C

    @functools.partial(
        pl.kernel,
        out_type=jax.ShapeDtypeStruct((NC, NP), jnp.float32),
        mesh=_sc_mesh(),
        scratch_types=[
            pltpu.VMEM((chc, C), jnp.int32),
            pltpu.VMEM((chc, C), jnp.int32),
            pltpu.VMEM((eptc,), jnp.float32),
            pltpu.VMEM_SHARED((NP,), jnp.float32),
        ],
    )
    def deg(psd_hbm, ew_hbm, zeros_hbm, out_hbm, psd_v, dst_v, ew_v, acc):
        c = lax.axis_index("c")
        s = lax.axis_index("s")
        rps = NP // NS
        pltpu.sync_copy(zeros_hbm.at[pl.ds(s * rps, rps)],
                        acc.at[pl.ds(s * rps, rps)])
        pltpu.sync_copy(psd_hbm.at[s, pl.ds(c * chc, chc)], psd_v)
        pltpu.sync_copy(ew_hbm.at[s, pl.ds(c * eptc, eptc)], ew_v)
        _unpack(psd_v, None, dst_v, chc)
        plsc.subcore_barrier()

        @pl.loop(0, chc)
        def _(j):
            pltpu.sync_copy(ew_v.at[pl.ds(j * C, C)],
                            acc.at[dst_v.at[j]], add=True)

        plsc.subcore_barrier()

        @pl.when(s == 0)
        def _():
            pltpu.sync_copy(acc, out_hbm.at[c])

    return deg


@functools.lru_cache(maxsize=None)
def _agg_kernel(dhc, ncp):
    @functools.partial(
        pl.kernel,
        out_type=jax.ShapeDtypeStruct((NC, ncp, NP, dhc), jnp.float32),
        mesh=_sc_mesh(),
        compiler_params=pltpu.CompilerParams(use_tc_tiling_on_sc=False),
        scratch_types=[
            pltpu.VMEM((CH // 2, C), jnp.int32),
            pltpu.VMEM((CH // 2, C), jnp.int32),
            pltpu.VMEM((CH // 2, C), jnp.int32),
            pltpu.VMEM((EPT // 2,), jnp.float32),
            pltpu.VMEM((C, dhc), jnp.float32),
            pltpu.VMEM((C, dhc), jnp.float32),
            pltpu.VMEM((C, dhc), jnp.float32),
            pltpu.VMEM((C, dhc), jnp.float32),
            pltpu.VMEM_SHARED((NP, dhc), jnp.float32),
            pltpu.VMEM_SHARED((NP, dhc), jnp.float32),
            pltpu.SemaphoreType.DMA,
            pltpu.SemaphoreType.DMA,
            pltpu.SemaphoreType.DMA,
            pltpu.SemaphoreType.DMA,
            pltpu.SemaphoreType.DMA,
            pltpu.SemaphoreType.DMA,
            pltpu.SemaphoreType.DMA,
            pltpu.SemaphoreType.DMA,
        ],
    )
    def agg(hp_hbm, psd_hbm, ew_hbm, zeros_hbm, out_hbm,
            psd_v, src_v, dst_v, ew_v, r0, r1, r2, r3, acc, hps,
            g0, g1, g2, g3, s0, s1, s2, s3):
        rows = (r0, r1, r2, r3)
        gsem = (g0, g1, g2, g3)
        ssem = (s0, s1, s2, s3)
        c = lax.axis_index("c")
        s = lax.axis_index("s")
        rps = NP // NS

        def scale(rows_v, j):
            @pl.loop(0, C // LANES)
            def _(g):
                ewv = ew_v[pl.ds(j * C + g * LANES, LANES)]
                for r16 in range(LANES):
                    coef = _splat(ewv, r16)
                    r = g * LANES + r16
                    for k in range(dhc // LANES):
                        sl = pl.ds(k * LANES, LANES)
                        rows_v[r, sl] = rows_v[r, sl] * coef

        def drain(sem, buf):
            pltpu.make_async_copy(zeros_hbm, buf, sem).wait()

        CH2 = CH // 2
        HF4 = CH2 // 4
        for cp in range(ncp):
            @pl.loop(0, rps // C)
            def _(k):
                pltpu.sync_copy(zeros_hbm,
                                acc.at[pl.ds(s * rps + k * C, C)])
            pltpu.sync_copy(hp_hbm.at[c * ncp + cp].at[pl.ds(s * rps, rps)],
                            hps.at[pl.ds(s * rps, rps)])
            plsc.subcore_barrier()

            for p in range(2):
                pltpu.sync_copy(psd_hbm.at[s, pl.ds(p * CH2, CH2)], psd_v)
                pltpu.sync_copy(ew_hbm.at[s, pl.ds(p * CH2 * C, CH2 * C)],
                                ew_v)
                _unpack(psd_v, src_v, dst_v, CH2)

                pltpu.async_copy(hps.at[src_v.at[0]], rows[0], gsem[0])
                pltpu.async_copy(hps.at[src_v.at[1]], rows[1], gsem[1])

                @pl.loop(0, HF4)
                def _(jj):
                    for b in range(4):
                        j = jj * 4 + b
                        b2 = (b + 2) % 4
                        jn = j + 2
                        if b < 2:
                            @pl.when(jj > 0)
                            def _():
                                drain(ssem[b2], rows[b2])
                            pltpu.async_copy(hps.at[src_v.at[jn]],
                                             rows[b2], gsem[b2])
                        else:
                            drain(ssem[b2], rows[b2])

                            @pl.when(jj < HF4 - 1)
                            def _():
                                pltpu.async_copy(hps.at[src_v.at[jn]],
                                                 rows[b2], gsem[b2])

                        drain(gsem[b], rows[b])
                        scale(rows[b], j)
                        pltpu.async_copy(rows[b], acc.at[dst_v.at[j]],
                                         ssem[b], add=True)

                drain(ssem[2], rows[2])
                drain(ssem[3], rows[3])

            plsc.subcore_barrier()

            @pl.when(s == 0)
            def _():
                pltpu.sync_copy(acc, out_hbm.at[c, cp])

            if cp + 1 < ncp:
                plsc.subcore_barrier()

    return agg


def _tc_first(degT, x, w1):
    d = w1.shape[1]
    G = _groups(d)
    dhc = d // G

    def body(deg_ref, x_ref, w_ref, dinv_ref, hp_ref):
        deg = deg_ref[:, 0] + deg_ref[:, 1] + 1.0
        dinv = jnp.where(deg > 0, lax.rsqrt(deg), 0.0)[:, None]
        dinv_ref[...] = dinv
        h = jnp.dot(x_ref[...], w_ref[...],
                    preferred_element_type=jnp.float32) * dinv
        for g in range(G):
            hp_ref[g] = h[:, g * dhc:(g + 1) * dhc]

    return pl.pallas_call(
        body,
        grid=(NP // RB,),
        in_specs=[
            pl.BlockSpec((RB, 2), lambda i: (i, 0)),
            pl.BlockSpec((RB, x.shape[1]), lambda i: (i, 0)),
            pl.BlockSpec(w1.shape, lambda i: (0, 0)),
        ],
        out_specs=[
            pl.BlockSpec((RB, 1), lambda i: (i, 0)),
            pl.BlockSpec((G, RB, dhc), lambda i: (0, i, 0)),
        ],
        out_shape=[
            jax.ShapeDtypeStruct((NP, 1), jnp.float32),
            jax.ShapeDtypeStruct((G, NP, dhc), jnp.float32),
        ],
    )(degT, x, w1)


def _tc_mid(agg, hp, dinv, b2d, wn):
    d = b2d.shape[1]
    G = agg.shape[0]
    dhc = agg.shape[2]
    dn = wn.shape[1]
    Gn = _groups(dn)
    dno = dn // Gn

    def body(a_ref, hp_ref, dinv_ref, b_ref, w_ref, out_ref):
        z = jnp.concatenate(
            [a_ref[g] + hp_ref[g] for g in range(G)], axis=1)
        dinv = dinv_ref[...]
        act = jnp.maximum(z * dinv + b_ref[...], 0.0)
        hn = jnp.dot(act, w_ref[...],
                     preferred_element_type=jnp.float32) * dinv
        for g in range(Gn):
            out_ref[g] = hn[:, g * dno:(g + 1) * dno]

    return pl.pallas_call(
        body,
        grid=(NP // RB,),
        in_specs=[
            pl.BlockSpec((G, RB, dhc), lambda i: (0, i, 0)),
            pl.BlockSpec((G, RB, dhc), lambda i: (0, i, 0)),
            pl.BlockSpec((RB, 1), lambda i: (i, 0)),
            pl.BlockSpec((1, d), lambda i: (0, 0)),
            pl.BlockSpec(wn.shape, lambda i: (0, 0)),
        ],
        out_specs=pl.BlockSpec((Gn, RB, dno), lambda i: (0, i, 0)),
        out_shape=jax.ShapeDtypeStruct((Gn, NP, dno), jnp.float32),
    )(agg, hp, dinv, b2d, wn)


def _tc_last(agg, hp, dinv, b2d, wf, bf2d):
    d = b2d.shape[1]
    G = agg.shape[0]
    dhc = agg.shape[2]
    dout = wf.shape[1]

    def body(a_ref, hp_ref, dinv_ref, b_ref, w_ref, bf_ref, out_ref):
        z = jnp.concatenate(
            [a_ref[g] + hp_ref[g] for g in range(G)], axis=1)
        dinv = dinv_ref[...]
        act = jnp.maximum(z * dinv + b_ref[...], 0.0)
        out_ref[...] = jnp.dot(act, w_ref[...],
                               preferred_element_type=jnp.float32) + bf_ref[...]

    return pl.pallas_call(
        body,
        grid=(NP // RB,),
        in_specs=[
            pl.BlockSpec((G, RB, dhc), lambda i: (0, i, 0)),
            pl.BlockSpec((G, RB, dhc), lambda i: (0, i, 0)),
            pl.BlockSpec((RB, 1), lambda i: (i, 0)),
            pl.BlockSpec((1, d), lambda i: (0, 0)),
            pl.BlockSpec(wf.shape, lambda i: (0, 0)),
            pl.BlockSpec((1, dout), lambda i: (0, 0)),
        ],
        out_specs=pl.BlockSpec((RB, dout), lambda i: (i, 0)),
        out_shape=jax.ShapeDtypeStruct((NP, dout), jnp.float32),
    )(agg, hp, dinv, b2d, wf, bf2d)


def _run_agg(hp, psd_r, ew_r):
    G, _, dhc = hp.shape
    ncp = G // NC
    out = _agg_kernel(dhc, ncp)(hp, psd_r, ew_r,
                                jnp.zeros((C, dhc), jnp.float32))
    return out.reshape(G, NP, dhc)


def kernel(x, edge_index, edge_attr, W1, b1, W2, b2, W3, b3, W4, b4, Wf, bf):
    pad = EP - E
    pad_idx = jnp.arange(pad, dtype=jnp.int32) % N
    src_f = jnp.concatenate([edge_index[0].astype(jnp.int32), pad_idx])
    dst_f = jnp.concatenate([edge_index[1].astype(jnp.int32), pad_idx])
    psd_r = ((src_f << 16) | dst_f).reshape(NS, CH, C)
    ew_r = jnp.concatenate(
        [edge_attr, jnp.zeros((pad,), edge_attr.dtype)]
    ).reshape(NS, EPT)
    xp = jnp.concatenate(
        [x, jnp.zeros((NP - N, x.shape[1]), x.dtype)], axis=0)

    deg_parts = _deg_kernel()(psd_r, ew_r, jnp.zeros((NP,), jnp.float32))
    degT = deg_parts.T

    dinv, hp = _tc_first(degT, xp, W1)

    for b_l, w_next in ((b1, W2), (b2, W3), (b3, W4)):
        agg = _run_agg(hp, psd_r, ew_r)
        hp = _tc_mid(agg, hp, dinv, b_l.reshape(1, -1), w_next)

    agg = _run_agg(hp, psd_r, ew_r)
    out = _tc_last(agg, hp, dinv, b4.reshape(1, -1), Wf, bf.reshape(1, -1))
    return out[:N]


# speedup vs baseline: 21.2648x; 1.0365x over previous
import functools

import jax
import jax.numpy as jnp
from jax import lax
from jax.experimental import pallas as pl
from jax.experimental.pallas import tpu as pltpu
from jax.experimental.pallas import tpu_sc as plsc

N = 10000
NP = 10240
E = 320000
NC, NS, LANES = 2, 16, 16
C = 128
CH = 160
EPT = CH * C
EP = NS * EPT
RB = 1280


def _groups(d):
    return 2 if d <= 64 else 4


def _sc_mesh():
    return plsc.VectorSubcoreMesh(core_axis_name="c", subcore_axis_name="s")


def _splat(vec, lane):
    idx = jnp.full((LANES, 1), lane, jnp.int32)
    dnums = lax.GatherDimensionNumbers(
        offset_dims=(), collapsed_slice_dims=(0,), start_index_map=(0,))
    return lax.gather(vec, idx, dnums, (1,),
                      mode=lax.GatherScatterMode.PROMISE_IN_BOUNDS)


def _unpack(psd_v, src_v, dst_v, nch):
    @pl.loop(0, nch)
    def _(j):
        @pl.loop(0, C // LANES)
        def _(g):
            sl = pl.ds(g * LANES, LANES)
            v = psd_v[j, sl]
            if src_v is not None:
                src_v[j, sl] = lax.shift_right_logical(v, 16)
            dst_v[j, sl] = lax.bitwise_and(v, 0xFFFF)


@functools.lru_cache(maxsize=None)
def _deg_kernel():
    chc = CH // NC
    eptc = chc * C

    @functools.partial(
        pl.kernel,
        out_type=jax.ShapeDtypeStruct((NC, NP), jnp.float32),
        mesh=_sc_mesh(),
        scratch_types=[
            pltpu.VMEM((chc, C), jnp.int32),
            pltpu.VMEM((chc, C), jnp.int32),
            pltpu.VMEM((eptc,), jnp.float32),
            pltpu.VMEM_SHARED((NP,), jnp.float32),
        ],
    )
    def deg(psd_hbm, ew_hbm, zeros_hbm, out_hbm, psd_v, dst_v, ew_v, acc):
        c = lax.axis_index("c")
        s = lax.axis_index("s")
        rps = NP // NS
        pltpu.sync_copy(zeros_hbm.at[pl.ds(s * rps, rps)],
                        acc.at[pl.ds(s * rps, rps)])
        pltpu.sync_copy(psd_hbm.at[s, pl.ds(c * chc, chc)], psd_v)
        pltpu.sync_copy(ew_hbm.at[s, pl.ds(c * eptc, eptc)], ew_v)
        _unpack(psd_v, None, dst_v, chc)
        plsc.subcore_barrier()

        @pl.loop(0, chc)
        def _(j):
            pltpu.sync_copy(ew_v.at[pl.ds(j * C, C)],
                            acc.at[dst_v.at[j]], add=True)

        plsc.subcore_barrier()

        @pl.when(s == 0)
        def _():
            pltpu.sync_copy(acc, out_hbm.at[c])

    return deg


@functools.lru_cache(maxsize=None)
def _agg_kernel(dhc, ncp):
    @functools.partial(
        pl.kernel,
        out_type=jax.ShapeDtypeStruct((NC, ncp, NP, dhc), jnp.float32),
        mesh=_sc_mesh(),
        compiler_params=pltpu.CompilerParams(use_tc_tiling_on_sc=False),
        scratch_types=[
            pltpu.VMEM((CH // 2, C), jnp.int32),
            pltpu.VMEM((CH // 2, C), jnp.int32),
            pltpu.VMEM((CH // 2, C), jnp.int32),
            pltpu.VMEM((EPT // 2,), jnp.float32),
            pltpu.VMEM((C, dhc), jnp.float32),
            pltpu.VMEM((C, dhc), jnp.float32),
            pltpu.VMEM((C, dhc), jnp.float32),
            pltpu.VMEM((C, dhc), jnp.float32),
            pltpu.VMEM_SHARED((NP, dhc), jnp.float32),
            pltpu.VMEM_SHARED((NP, dhc), jnp.float32),
            pltpu.SemaphoreType.DMA,
            pltpu.SemaphoreType.DMA,
            pltpu.SemaphoreType.DMA,
            pltpu.SemaphoreType.DMA,
            pltpu.SemaphoreType.DMA,
            pltpu.SemaphoreType.DMA,
            pltpu.SemaphoreType.DMA,
            pltpu.SemaphoreType.DMA,
            pltpu.SemaphoreType.DMA,
            pltpu.SemaphoreType.DMA,
            pltpu.SemaphoreType.DMA,
        ],
    )
    def agg(hp_hbm, psd_hbm, ew_hbm, zeros_hbm, out_hbm,
            psd_v, src_v, dst_v, ew_v, r0, r1, r2, r3, acc, hps,
            g0, g1, g2, g3, s0, s1, s2, s3, z0, z1, z2):
        rows = (r0, r1, r2, r3)
        gsem = (g0, g1, g2, g3)
        ssem = (s0, s1, s2, s3)
        c = lax.axis_index("c")
        s = lax.axis_index("s")
        rps = NP // NS

        def scale(rows_v, j):
            @pl.loop(0, C // LANES)
            def _(g):
                ewv = ew_v[pl.ds(j * C + g * LANES, LANES)]
                for r16 in range(LANES):
                    coef = _splat(ewv, r16)
                    r = g * LANES + r16
                    for k in range(dhc // LANES):
                        sl = pl.ds(k * LANES, LANES)
                        rows_v[r, sl] = rows_v[r, sl] * coef

        def drain(sem, buf):
            pltpu.make_async_copy(zeros_hbm, buf, sem).wait()

        CH2 = CH // 2
        HF4 = CH2 // 4
        for cp in range(ncp):
            for k in range(rps // C):
                pltpu.async_copy(zeros_hbm,
                                 acc.at[pl.ds(s * rps + k * C, C)], z0)
            pltpu.async_copy(hp_hbm.at[c * ncp + cp].at[pl.ds(s * rps, rps)],
                             hps.at[pl.ds(s * rps, rps)], z0)
            pltpu.async_copy(psd_hbm.at[s, pl.ds(0, CH2)], psd_v, z1)
            pltpu.async_copy(ew_hbm.at[s, pl.ds(0, CH2 * C)], ew_v, z2)
            pltpu.make_async_copy(psd_hbm.at[s, pl.ds(0, CH2)],
                                  psd_v, z1).wait()
            _unpack(psd_v, src_v, dst_v, CH2)
            for k in range(rps // C):
                pltpu.make_async_copy(
                    zeros_hbm, acc.at[pl.ds(s * rps + k * C, C)], z0).wait()
            pltpu.make_async_copy(
                hp_hbm.at[c * ncp + cp].at[pl.ds(s * rps, rps)],
                hps.at[pl.ds(s * rps, rps)], z0).wait()
            pltpu.make_async_copy(ew_hbm.at[s, pl.ds(0, CH2 * C)],
                                  ew_v, z2).wait()
            plsc.subcore_barrier()

            for p in range(2):
                if p == 0:
                    pltpu.async_copy(psd_hbm.at[s, pl.ds(CH2, CH2)],
                                     psd_v, z1)
                else:
                    pltpu.async_copy(ew_hbm.at[s, pl.ds(CH2 * C, CH2 * C)],
                                     ew_v, z2)
                    pltpu.make_async_copy(psd_hbm.at[s, pl.ds(CH2, CH2)],
                                          psd_v, z1).wait()
                    _unpack(psd_v, src_v, dst_v, CH2)
                    pltpu.make_async_copy(
                        ew_hbm.at[s, pl.ds(CH2 * C, CH2 * C)],
                        ew_v, z2).wait()

                pltpu.async_copy(hps.at[src_v.at[0]], rows[0], gsem[0])
                pltpu.async_copy(hps.at[src_v.at[1]], rows[1], gsem[1])

                @pl.loop(0, HF4)
                def _(jj):
                    for b in range(4):
                        j = jj * 4 + b
                        b2 = (b + 2) % 4
                        jn = j + 2
                        if b < 2:
                            @pl.when(jj > 0)
                            def _():
                                drain(ssem[b2], rows[b2])
                            pltpu.async_copy(hps.at[src_v.at[jn]],
                                             rows[b2], gsem[b2])
                        else:
                            drain(ssem[b2], rows[b2])

                            @pl.when(jj < HF4 - 1)
                            def _():
                                pltpu.async_copy(hps.at[src_v.at[jn]],
                                                 rows[b2], gsem[b2])

                        drain(gsem[b], rows[b])
                        scale(rows[b], j)
                        pltpu.async_copy(rows[b], acc.at[dst_v.at[j]],
                                         ssem[b], add=True)

                drain(ssem[2], rows[2])
                drain(ssem[3], rows[3])

            plsc.subcore_barrier()

            @pl.when(s == 0)
            def _():
                pltpu.sync_copy(acc, out_hbm.at[c, cp])

            if cp + 1 < ncp:
                plsc.subcore_barrier()

    return agg


def _tc_first(degT, x, w1):
    d = w1.shape[1]
    G = _groups(d)
    dhc = d // G

    def body(deg_ref, x_ref, w_ref, dinv_ref, hp_ref):
        deg = deg_ref[:, 0] + deg_ref[:, 1] + 1.0
        dinv = jnp.where(deg > 0, lax.rsqrt(deg), 0.0)[:, None]
        dinv_ref[...] = dinv
        h = jnp.dot(x_ref[...], w_ref[...],
                    preferred_element_type=jnp.float32) * dinv
        for g in range(G):
            hp_ref[g] = h[:, g * dhc:(g + 1) * dhc]

    return pl.pallas_call(
        body,
        grid=(NP // RB,),
        in_specs=[
            pl.BlockSpec((RB, 2), lambda i: (i, 0)),
            pl.BlockSpec((RB, x.shape[1]), lambda i: (i, 0)),
            pl.BlockSpec(w1.shape, lambda i: (0, 0)),
        ],
        out_specs=[
            pl.BlockSpec((RB, 1), lambda i: (i, 0)),
            pl.BlockSpec((G, RB, dhc), lambda i: (0, i, 0)),
        ],
        out_shape=[
            jax.ShapeDtypeStruct((NP, 1), jnp.float32),
            jax.ShapeDtypeStruct((G, NP, dhc), jnp.float32),
        ],
    )(degT, x, w1)


def _tc_mid(agg, hp, dinv, b2d, wn):
    d = b2d.shape[1]
    G = agg.shape[0]
    dhc = agg.shape[2]
    dn = wn.shape[1]
    Gn = _groups(dn)
    dno = dn // Gn

    def body(a_ref, hp_ref, dinv_ref, b_ref, w_ref, out_ref):
        z = jnp.concatenate(
            [a_ref[g] + hp_ref[g] for g in range(G)], axis=1)
        dinv = dinv_ref[...]
        act = jnp.maximum(z * dinv + b_ref[...], 0.0)
        hn = jnp.dot(act, w_ref[...],
                     preferred_element_type=jnp.float32) * dinv
        for g in range(Gn):
            out_ref[g] = hn[:, g * dno:(g + 1) * dno]

    return pl.pallas_call(
        body,
        grid=(NP // RB,),
        in_specs=[
            pl.BlockSpec((G, RB, dhc), lambda i: (0, i, 0)),
            pl.BlockSpec((G, RB, dhc), lambda i: (0, i, 0)),
            pl.BlockSpec((RB, 1), lambda i: (i, 0)),
            pl.BlockSpec((1, d), lambda i: (0, 0)),
            pl.BlockSpec(wn.shape, lambda i: (0, 0)),
        ],
        out_specs=pl.BlockSpec((Gn, RB, dno), lambda i: (0, i, 0)),
        out_shape=jax.ShapeDtypeStruct((Gn, NP, dno), jnp.float32),
    )(agg, hp, dinv, b2d, wn)


def _tc_last(agg, hp, dinv, b2d, wf, bf2d):
    d = b2d.shape[1]
    G = agg.shape[0]
    dhc = agg.shape[2]
    dout = wf.shape[1]

    def body(a_ref, hp_ref, dinv_ref, b_ref, w_ref, bf_ref, out_ref):
        z = jnp.concatenate(
            [a_ref[g] + hp_ref[g] for g in range(G)], axis=1)
        dinv = dinv_ref[...]
        act = jnp.maximum(z * dinv + b_ref[...], 0.0)
        out_ref[...] = jnp.dot(act, w_ref[...],
                               preferred_element_type=jnp.float32) + bf_ref[...]

    return pl.pallas_call(
        body,
        grid=(NP // RB,),
        in_specs=[
            pl.BlockSpec((G, RB, dhc), lambda i: (0, i, 0)),
            pl.BlockSpec((G, RB, dhc), lambda i: (0, i, 0)),
            pl.BlockSpec((RB, 1), lambda i: (i, 0)),
            pl.BlockSpec((1, d), lambda i: (0, 0)),
            pl.BlockSpec(wf.shape, lambda i: (0, 0)),
            pl.BlockSpec((1, dout), lambda i: (0, 0)),
        ],
        out_specs=pl.BlockSpec((RB, dout), lambda i: (i, 0)),
        out_shape=jax.ShapeDtypeStruct((NP, dout), jnp.float32),
    )(agg, hp, dinv, b2d, wf, bf2d)


def _run_agg(hp, psd_r, ew_r):
    G, _, dhc = hp.shape
    ncp = G // NC
    out = _agg_kernel(dhc, ncp)(hp, psd_r, ew_r,
                                jnp.zeros((C, dhc), jnp.float32))
    return out.reshape(G, NP, dhc)


def kernel(x, edge_index, edge_attr, W1, b1, W2, b2, W3, b3, W4, b4, Wf, bf):
    pad = EP - E
    pad_idx = jnp.arange(pad, dtype=jnp.int32) % N
    src_f = jnp.concatenate([edge_index[0].astype(jnp.int32), pad_idx])
    dst_f = jnp.concatenate([edge_index[1].astype(jnp.int32), pad_idx])
    psd_r = ((src_f << 16) | dst_f).reshape(NS, CH, C)
    ew_r = jnp.concatenate(
        [edge_attr, jnp.zeros((pad,), edge_attr.dtype)]
    ).reshape(NS, EPT)
    xp = jnp.concatenate(
        [x, jnp.zeros((NP - N, x.shape[1]), x.dtype)], axis=0)

    deg_parts = _deg_kernel()(psd_r, ew_r, jnp.zeros((NP,), jnp.float32))
    degT = deg_parts.T

    dinv, hp = _tc_first(degT, xp, W1)

    for b_l, w_next in ((b1, W2), (b2, W3), (b3, W4)):
        agg = _run_agg(hp, psd_r, ew_r)
        hp = _tc_mid(agg, hp, dinv, b_l.reshape(1, -1), w_next)

    agg = _run_agg(hp, psd_r, ew_r)
    out = _tc_last(agg, hp, dinv, b4.reshape(1, -1), Wf, bf.reshape(1, -1))
    return out[:N]


# speedup vs baseline: 21.4897x; 1.0106x over previous
import functools

import jax
import jax.numpy as jnp
from jax import lax
from jax.experimental import pallas as pl
from jax.experimental.pallas import tpu as pltpu
from jax.experimental.pallas import tpu_sc as plsc

N = 10000
NP = 10240
E = 320000
NC, NS, LANES = 2, 16, 16
C = 128
CH = 160
EPT = CH * C
EP = NS * EPT
RB = 1280


def _groups(d):
    return 2 if d <= 64 else 4


def _sc_mesh():
    return plsc.VectorSubcoreMesh(core_axis_name="c", subcore_axis_name="s")


def _splat(vec, lane):
    idx = jnp.full((LANES, 1), lane, jnp.int32)
    dnums = lax.GatherDimensionNumbers(
        offset_dims=(), collapsed_slice_dims=(0,), start_index_map=(0,))
    return lax.gather(vec, idx, dnums, (1,),
                      mode=lax.GatherScatterMode.PROMISE_IN_BOUNDS)


def _unpack(psd_v, src_v, dst_v, nch):
    @pl.loop(0, nch)
    def _(j):
        @pl.loop(0, C // LANES)
        def _(g):
            sl = pl.ds(g * LANES, LANES)
            v = psd_v[j, sl]
            if src_v is not None:
                src_v[j, sl] = lax.shift_right_logical(v, 16)
            dst_v[j, sl] = lax.bitwise_and(v, 0xFFFF)


@functools.lru_cache(maxsize=None)
def _deg_kernel():
    chc = CH // NC
    eptc = chc * C

    @functools.partial(
        pl.kernel,
        out_type=jax.ShapeDtypeStruct((NC, NP), jnp.float32),
        mesh=_sc_mesh(),
        scratch_types=[
            pltpu.VMEM((chc, C), jnp.int32),
            pltpu.VMEM((chc, C), jnp.int32),
            pltpu.VMEM((eptc,), jnp.float32),
            pltpu.VMEM_SHARED((NP,), jnp.float32),
            pltpu.SemaphoreType.DMA,
            pltpu.SemaphoreType.DMA,
        ],
    )
    def deg(psd_hbm, ew_hbm, zeros_hbm, out_hbm, psd_v, dst_v, ew_v, acc,
            z0, z1):
        c = lax.axis_index("c")
        s = lax.axis_index("s")
        rps = NP // NS
        pltpu.async_copy(zeros_hbm.at[pl.ds(s * rps, rps)],
                         acc.at[pl.ds(s * rps, rps)], z0)
        pltpu.async_copy(ew_hbm.at[s, pl.ds(c * eptc, eptc)], ew_v, z0)
        pltpu.async_copy(psd_hbm.at[s, pl.ds(c * chc, chc)], psd_v, z1)
        pltpu.make_async_copy(psd_hbm.at[s, pl.ds(c * chc, chc)],
                              psd_v, z1).wait()
        _unpack(psd_v, None, dst_v, chc)
        pltpu.make_async_copy(zeros_hbm.at[pl.ds(s * rps, rps)],
                              acc.at[pl.ds(s * rps, rps)], z0).wait()
        pltpu.make_async_copy(ew_hbm.at[s, pl.ds(c * eptc, eptc)],
                              ew_v, z0).wait()
        plsc.subcore_barrier()

        @pl.loop(0, chc)
        def _(j):
            pltpu.sync_copy(ew_v.at[pl.ds(j * C, C)],
                            acc.at[dst_v.at[j]], add=True)

        plsc.subcore_barrier()

        @pl.when(s == 0)
        def _():
            pltpu.sync_copy(acc, out_hbm.at[c])

    return deg


@functools.lru_cache(maxsize=None)
def _agg_kernel(dhc, ncp):
    @functools.partial(
        pl.kernel,
        out_type=jax.ShapeDtypeStruct((NC, ncp, NP, dhc), jnp.float32),
        mesh=_sc_mesh(),
        compiler_params=pltpu.CompilerParams(use_tc_tiling_on_sc=False),
        scratch_types=[
            pltpu.VMEM((CH // 2, C), jnp.int32),
            pltpu.VMEM((CH // 2, C), jnp.int32),
            pltpu.VMEM((CH // 2, C), jnp.int32),
            pltpu.VMEM((EPT // 2,), jnp.float32),
            pltpu.VMEM((C, dhc), jnp.float32),
            pltpu.VMEM((C, dhc), jnp.float32),
            pltpu.VMEM((C, dhc), jnp.float32),
            pltpu.VMEM((C, dhc), jnp.float32),
            pltpu.VMEM_SHARED((NP, dhc), jnp.float32),
            pltpu.VMEM_SHARED((NP, dhc), jnp.float32),
            pltpu.SemaphoreType.DMA,
            pltpu.SemaphoreType.DMA,
            pltpu.SemaphoreType.DMA,
            pltpu.SemaphoreType.DMA,
            pltpu.SemaphoreType.DMA,
            pltpu.SemaphoreType.DMA,
            pltpu.SemaphoreType.DMA,
            pltpu.SemaphoreType.DMA,
            pltpu.SemaphoreType.DMA,
            pltpu.SemaphoreType.DMA,
            pltpu.SemaphoreType.DMA,
        ],
    )
    def agg(hp_hbm, psd_hbm, ew_hbm, zeros_hbm, out_hbm,
            psd_v, src_v, dst_v, ew_v, r0, r1, r2, r3, acc, hps,
            g0, g1, g2, g3, s0, s1, s2, s3, z0, z1, z2):
        rows = (r0, r1, r2, r3)
        gsem = (g0, g1, g2, g3)
        ssem = (s0, s1, s2, s3)
        c = lax.axis_index("c")
        s = lax.axis_index("s")
        rps = NP // NS

        def scale(rows_v, j):
            @pl.loop(0, C // LANES)
            def _(g):
                ewv = ew_v[pl.ds(j * C + g * LANES, LANES)]
                for r16 in range(LANES):
                    coef = _splat(ewv, r16)
                    r = g * LANES + r16
                    for k in range(dhc // LANES):
                        sl = pl.ds(k * LANES, LANES)
                        rows_v[r, sl] = rows_v[r, sl] * coef

        def drain(sem, buf):
            pltpu.make_async_copy(zeros_hbm, buf, sem).wait()

        CH2 = CH // 2
        HF4 = CH2 // 4
        for cp in range(ncp):
            for k in range(rps // C):
                pltpu.async_copy(zeros_hbm,
                                 acc.at[pl.ds(s * rps + k * C, C)], z0)
            pltpu.async_copy(hp_hbm.at[c * ncp + cp].at[pl.ds(s * rps, rps)],
                             hps.at[pl.ds(s * rps, rps)], z0)
            pltpu.async_copy(psd_hbm.at[s, pl.ds(0, CH2)], psd_v, z1)
            pltpu.async_copy(ew_hbm.at[s, pl.ds(0, CH2 * C)], ew_v, z2)
            pltpu.make_async_copy(psd_hbm.at[s, pl.ds(0, CH2)],
                                  psd_v, z1).wait()
            _unpack(psd_v, src_v, dst_v, CH2)
            for k in range(rps // C):
                pltpu.make_async_copy(
                    zeros_hbm, acc.at[pl.ds(s * rps + k * C, C)], z0).wait()
            pltpu.make_async_copy(
                hp_hbm.at[c * ncp + cp].at[pl.ds(s * rps, rps)],
                hps.at[pl.ds(s * rps, rps)], z0).wait()
            pltpu.make_async_copy(ew_hbm.at[s, pl.ds(0, CH2 * C)],
                                  ew_v, z2).wait()
            plsc.subcore_barrier()

            for p in range(2):
                if p == 0:
                    pltpu.async_copy(psd_hbm.at[s, pl.ds(CH2, CH2)],
                                     psd_v, z1)
                else:
                    pltpu.async_copy(ew_hbm.at[s, pl.ds(CH2 * C, CH2 * C)],
                                     ew_v, z2)
                    pltpu.make_async_copy(psd_hbm.at[s, pl.ds(CH2, CH2)],
                                          psd_v, z1).wait()
                    _unpack(psd_v, src_v, dst_v, CH2)
                    pltpu.make_async_copy(
                        ew_hbm.at[s, pl.ds(CH2 * C, CH2 * C)],
                        ew_v, z2).wait()

                pltpu.async_copy(hps.at[src_v.at[0]], rows[0], gsem[0])
                pltpu.async_copy(hps.at[src_v.at[1]], rows[1], gsem[1])

                @pl.loop(0, HF4)
                def _(jj):
                    for b in range(4):
                        j = jj * 4 + b
                        b2 = (b + 2) % 4
                        jn = j + 2
                        if b < 2:
                            @pl.when(jj > 0)
                            def _():
                                drain(ssem[b2], rows[b2])
                            pltpu.async_copy(hps.at[src_v.at[jn]],
                                             rows[b2], gsem[b2])
                        else:
                            drain(ssem[b2], rows[b2])

                            @pl.when(jj < HF4 - 1)
                            def _():
                                pltpu.async_copy(hps.at[src_v.at[jn]],
                                                 rows[b2], gsem[b2])

                        drain(gsem[b], rows[b])
                        scale(rows[b], j)
                        pltpu.async_copy(rows[b], acc.at[dst_v.at[j]],
                                         ssem[b], add=True)

                drain(ssem[2], rows[2])
                drain(ssem[3], rows[3])

            plsc.subcore_barrier()

            @pl.when(s == 0)
            def _():
                pltpu.sync_copy(acc, out_hbm.at[c, cp])

            if cp + 1 < ncp:
                plsc.subcore_barrier()

    return agg


def _tc_first(degT, x, w1):
    d = w1.shape[1]
    G = _groups(d)
    dhc = d // G

    def body(deg_ref, x_ref, w_ref, dinv_ref, hp_ref):
        deg = deg_ref[0] + deg_ref[1] + 1.0
        dinv = jnp.where(deg > 0, lax.rsqrt(deg), 0.0)[:, None]
        dinv_ref[...] = dinv
        h = jnp.dot(x_ref[...], w_ref[...],
                    preferred_element_type=jnp.float32) * dinv
        for g in range(G):
            hp_ref[g] = h[:, g * dhc:(g + 1) * dhc]

    return pl.pallas_call(
        body,
        grid=(NP // RB,),
        in_specs=[
            pl.BlockSpec((2, RB), lambda i: (0, i)),
            pl.BlockSpec((RB, x.shape[1]), lambda i: (i, 0)),
            pl.BlockSpec(w1.shape, lambda i: (0, 0)),
        ],
        out_specs=[
            pl.BlockSpec((RB, 1), lambda i: (i, 0)),
            pl.BlockSpec((G, RB, dhc), lambda i: (0, i, 0)),
        ],
        out_shape=[
            jax.ShapeDtypeStruct((NP, 1), jnp.float32),
            jax.ShapeDtypeStruct((G, NP, dhc), jnp.float32),
        ],
    )(degT, x, w1)


def _tc_mid(agg, hp, dinv, b2d, wn):
    d = b2d.shape[1]
    G = agg.shape[0]
    dhc = agg.shape[2]
    dn = wn.shape[1]
    Gn = _groups(dn)
    dno = dn // Gn

    def body(a_ref, hp_ref, dinv_ref, b_ref, w_ref, out_ref):
        z = jnp.concatenate(
            [a_ref[g] + hp_ref[g] for g in range(G)], axis=1)
        dinv = dinv_ref[...]
        act = jnp.maximum(z * dinv + b_ref[...], 0.0)
        hn = jnp.dot(act, w_ref[...],
                     preferred_element_type=jnp.float32) * dinv
        for g in range(Gn):
            out_ref[g] = hn[:, g * dno:(g + 1) * dno]

    return pl.pallas_call(
        body,
        grid=(NP // RB,),
        in_specs=[
            pl.BlockSpec((G, RB, dhc), lambda i: (0, i, 0)),
            pl.BlockSpec((G, RB, dhc), lambda i: (0, i, 0)),
            pl.BlockSpec((RB, 1), lambda i: (i, 0)),
            pl.BlockSpec((1, d), lambda i: (0, 0)),
            pl.BlockSpec(wn.shape, lambda i: (0, 0)),
        ],
        out_specs=pl.BlockSpec((Gn, RB, dno), lambda i: (0, i, 0)),
        out_shape=jax.ShapeDtypeStruct((Gn, NP, dno), jnp.float32),
    )(agg, hp, dinv, b2d, wn)


def _tc_last(agg, hp, dinv, b2d, wf, bf2d):
    d = b2d.shape[1]
    G = agg.shape[0]
    dhc = agg.shape[2]
    dout = wf.shape[1]

    def body(a_ref, hp_ref, dinv_ref, b_ref, w_ref, bf_ref, out_ref):
        z = jnp.concatenate(
            [a_ref[g] + hp_ref[g] for g in range(G)], axis=1)
        dinv = dinv_ref[...]
        act = jnp.maximum(z * dinv + b_ref[...], 0.0)
        out_ref[...] = jnp.dot(act, w_ref[...],
                               preferred_element_type=jnp.float32) + bf_ref[...]

    return pl.pallas_call(
        body,
        grid=(NP // RB,),
        in_specs=[
            pl.BlockSpec((G, RB, dhc), lambda i: (0, i, 0)),
            pl.BlockSpec((G, RB, dhc), lambda i: (0, i, 0)),
            pl.BlockSpec((RB, 1), lambda i: (i, 0)),
            pl.BlockSpec((1, d), lambda i: (0, 0)),
            pl.BlockSpec(wf.shape, lambda i: (0, 0)),
            pl.BlockSpec((1, dout), lambda i: (0, 0)),
        ],
        out_specs=pl.BlockSpec((RB, dout), lambda i: (i, 0)),
        out_shape=jax.ShapeDtypeStruct((NP, dout), jnp.float32),
    )(agg, hp, dinv, b2d, wf, bf2d)


def _run_agg(hp, psd_r, ew_r):
    G, _, dhc = hp.shape
    ncp = G // NC
    out = _agg_kernel(dhc, ncp)(hp, psd_r, ew_r,
                                jnp.zeros((C, dhc), jnp.float32))
    return out.reshape(G, NP, dhc)


def kernel(x, edge_index, edge_attr, W1, b1, W2, b2, W3, b3, W4, b4, Wf, bf):
    pad = EP - E
    pad_idx = jnp.arange(pad, dtype=jnp.int32) % N
    src_f = jnp.concatenate([edge_index[0].astype(jnp.int32), pad_idx])
    dst_f = jnp.concatenate([edge_index[1].astype(jnp.int32), pad_idx])
    psd_r = ((src_f << 16) | dst_f).reshape(NS, CH, C)
    ew_r = jnp.concatenate(
        [edge_attr, jnp.zeros((pad,), edge_attr.dtype)]
    ).reshape(NS, EPT)
    xp = jnp.concatenate(
        [x, jnp.zeros((NP - N, x.shape[1]), x.dtype)], axis=0)

    deg_parts = _deg_kernel()(psd_r, ew_r, jnp.zeros((NP,), jnp.float32))

    dinv, hp = _tc_first(deg_parts, xp, W1)

    for b_l, w_next in ((b1, W2), (b2, W3), (b3, W4)):
        agg = _run_agg(hp, psd_r, ew_r)
        hp = _tc_mid(agg, hp, dinv, b_l.reshape(1, -1), w_next)

    agg = _run_agg(hp, psd_r, ew_r)
    out = _tc_last(agg, hp, dinv, b4.reshape(1, -1), Wf, bf.reshape(1, -1))
    return out[:N]
